# Initial kernel scaffold; baseline (speedup 1.0000x reference)
import jax
import jax.numpy as jnp
from jax.experimental import pallas as pl


def kernel(z, one_hot, W):
    raise NotImplementedError("write your pallas kernel here")


# speedup vs baseline: 4.9842x; 4.9842x over previous
import jax
import jax.numpy as jnp
from jax.experimental import pallas as pl
from jax.experimental.pallas import tpu as pltpu

_N_E = 1792
_E_DIM = 256
_BETA = 0.25
_NB = _N_E // 7
_BATCH = 64
_SEQ = 1024
_ROWS = _BATCH * _SEQ


def _vq_body(one_hot_ref, z_ref, w_ref,
             zqt_ref, loss_ref, perp_ref, enc_ref, idx_ref,
             loss_acc, cnt_acc):
    b = pl.program_id(0)

    def _amax(j, carry):
        bv, bi = carry
        v = one_hot_ref[j]
        take = v > bv
        return jnp.where(take, v, bv), jnp.where(take, j, bi)

    _, pos = jax.lax.fori_loop(1, 7, _amax, (one_hot_ref[0], jnp.int32(0)))

    w = w_ref[pl.ds(pos * _NB, _NB), :]
    wt = w.T

    z2 = z_ref[0]
    scores = jax.lax.dot_general(z2, w, (((1,), (1,)), ((), ())),
                                 preferred_element_type=jnp.float32)
    znorm = jnp.sum(z2 * z2, axis=1, keepdims=True)
    wnorm = jnp.sum(wt * wt, axis=0, keepdims=True)
    d = znorm + wnorm - 2.0 * scores

    mins = jnp.min(d, axis=1, keepdims=True)
    lane = jax.lax.broadcasted_iota(jnp.int32, d.shape, 1)
    idx = jnp.min(jnp.where(d == mins, lane, _NB), axis=1, keepdims=True)
    onehot = (lane == idx).astype(jnp.float32)

    enc_ref[...] = onehot
    idx_ref[...] = idx
    zqt_ref[0] = jax.lax.dot_general(wt, onehot, (((1,), (1,)), ((), ())),
                                     preferred_element_type=jnp.float32)

    @pl.when(b == 0)
    def _init():
        loss_acc[0, 0] = 0.0
        cnt_acc[...] = jnp.zeros_like(cnt_acc)

    loss_acc[0, 0] += jnp.sum(mins)
    cnt_acc[...] += jnp.sum(onehot, axis=0, keepdims=True)

    @pl.when(b == _BATCH - 1)
    def _fini():
        loss = (1.0 + _BETA) * loss_acc[0, 0] / (_ROWS * _E_DIM)
        loss_ref[...] = jnp.reshape(loss, (1, 1))
        e_mean = cnt_acc[...] * (1.0 / _ROWS)
        ent = jnp.sum(e_mean * jnp.log(e_mean + 1e-10))
        perp_ref[...] = jnp.reshape(jnp.exp(-ent), (1, 1))


def kernel(z, one_hot, W):
    zqt, loss, perp, enc, idx = pl.pallas_call(
        _vq_body,
        grid=(_BATCH,),
        in_specs=[
            pl.BlockSpec(memory_space=pltpu.SMEM),
            pl.BlockSpec((1, _SEQ, _E_DIM), lambda b: (b, 0, 0)),
            pl.BlockSpec((_N_E, _E_DIM), lambda b: (0, 0)),
        ],
        out_specs=[
            pl.BlockSpec((1, _E_DIM, _SEQ), lambda b: (b, 0, 0)),
            pl.BlockSpec((1, 1), lambda b: (0, 0)),
            pl.BlockSpec((1, 1), lambda b: (0, 0)),
            pl.BlockSpec((_SEQ, _NB), lambda b: (b, 0)),
            pl.BlockSpec((_SEQ, 1), lambda b: (b, 0)),
        ],
        out_shape=[
            jax.ShapeDtypeStruct((_BATCH, _E_DIM, _SEQ), jnp.float32),
            jax.ShapeDtypeStruct((1, 1), jnp.float32),
            jax.ShapeDtypeStruct((1, 1), jnp.float32),
            jax.ShapeDtypeStruct((_ROWS, _NB), jnp.float32),
            jax.ShapeDtypeStruct((_ROWS, 1), jnp.int32),
        ],
        scratch_shapes=[
            pltpu.SMEM((1, 1), jnp.float32),
            pltpu.VMEM((1, _NB), jnp.float32),
        ],
    )(one_hot, z, W)
    return (zqt, loss.reshape(()), (perp.reshape(()), enc, idx))


# speedup vs baseline: 6.1334x; 1.2306x over previous
import jax
import jax.numpy as jnp
from jax.experimental import pallas as pl
from jax.experimental.pallas import tpu as pltpu

_N_E = 1792
_E_DIM = 256
_BETA = 0.25
_NB = _N_E // 7
_BATCH = 64
_SEQ = 1024
_ROWS = _BATCH * _SEQ
_BPG = 2
_STEPS = _BATCH // _BPG
_BR = _BPG * _SEQ


def _vq_body(one_hot_ref, z_ref, w_ref,
             zqt_ref, loss_ref, perp_ref, enc_ref, idx_ref,
             loss_acc, cnt_acc):
    b = pl.program_id(0)

    def _amax(j, carry):
        bv, bi = carry
        v = one_hot_ref[j]
        take = v > bv
        return jnp.where(take, v, bv), jnp.where(take, j, bi)

    _, pos = jax.lax.fori_loop(1, 7, _amax, (one_hot_ref[0], jnp.int32(0)))

    w = w_ref[pl.ds(pos * _NB, _NB), :]
    wt = w.T

    z2 = z_ref[...].reshape(_BR, _E_DIM)
    scores = jax.lax.dot_general(z2, w, (((1,), (1,)), ((), ())),
                                 preferred_element_type=jnp.float32)
    znorm = jnp.sum(z2 * z2, axis=1, keepdims=True)
    wnorm = jnp.sum(wt * wt, axis=0, keepdims=True)
    d = znorm + wnorm - 2.0 * scores

    mins = jnp.min(d, axis=1, keepdims=True)
    lane = jax.lax.broadcasted_iota(jnp.int32, d.shape, 1)
    idx = jnp.min(jnp.where(d == mins, lane, _NB), axis=1, keepdims=True)
    onehot = (lane == idx).astype(jnp.float32)

    enc_ref[...] = onehot
    idx_ref[...] = idx
    for bi in range(_BPG):
        oh = onehot[bi * _SEQ:(bi + 1) * _SEQ, :]
        zqt_ref[bi] = jax.lax.dot_general(wt, oh, (((1,), (1,)), ((), ())),
                                          preferred_element_type=jnp.float32)

    @pl.when(b == 0)
    def _init():
        loss_acc[0, 0] = 0.0
        cnt_acc[...] = jnp.zeros_like(cnt_acc)

    loss_acc[0, 0] += jnp.sum(mins)
    cnt_acc[...] += jnp.sum(onehot, axis=0, keepdims=True)

    @pl.when(b == _STEPS - 1)
    def _fini():
        loss = (1.0 + _BETA) * loss_acc[0, 0] / (_ROWS * _E_DIM)
        loss_ref[...] = jnp.reshape(loss, (1, 1))
        e_mean = cnt_acc[...] * (1.0 / _ROWS)
        ent = jnp.sum(e_mean * jnp.log(e_mean + 1e-10))
        perp_ref[...] = jnp.reshape(jnp.exp(-ent), (1, 1))


def kernel(z, one_hot, W):
    zqt, loss, perp, enc, idx = pl.pallas_call(
        _vq_body,
        grid=(_STEPS,),
        in_specs=[
            pl.BlockSpec(memory_space=pltpu.SMEM),
            pl.BlockSpec((_BPG, _SEQ, _E_DIM), lambda b: (b, 0, 0)),
            pl.BlockSpec((_N_E, _E_DIM), lambda b: (0, 0)),
        ],
        out_specs=[
            pl.BlockSpec((_BPG, _E_DIM, _SEQ), lambda b: (b, 0, 0)),
            pl.BlockSpec((1, 1), lambda b: (0, 0)),
            pl.BlockSpec((1, 1), lambda b: (0, 0)),
            pl.BlockSpec((_BR, _NB), lambda b: (b, 0)),
            pl.BlockSpec((_BR, 1), lambda b: (b, 0)),
        ],
        out_shape=[
            jax.ShapeDtypeStruct((_BATCH, _E_DIM, _SEQ), jnp.float32),
            jax.ShapeDtypeStruct((1, 1), jnp.float32),
            jax.ShapeDtypeStruct((1, 1), jnp.float32),
            jax.ShapeDtypeStruct((_ROWS, _NB), jnp.float32),
            jax.ShapeDtypeStruct((_ROWS, 1), jnp.int32),
        ],
        scratch_shapes=[
            pltpu.SMEM((1, 1), jnp.float32),
            pltpu.VMEM((1, _NB), jnp.float32),
        ],
    )(one_hot, z, W)
    return (zqt, loss.reshape(()), (perp.reshape(()), enc, idx))
